# baseline (speedup 1.0000x reference)
import functools

import jax
import jax.numpy as jnp
from jax import lax
from jax.experimental import pallas as pl
from jax.experimental.pallas import tpu as pltpu

N = 10000
E = 640000
T = 50
NF = 32
NC = 2

FC_BK = 6400
FC_STEPS = (NF * N) // FC_BK


def _fc_head_body(flat_ref, w1_ref, b1_ref, w2_ref, b2_ref, w3_ref, b3_ref,
                  out_ref, acc_ref):
    step = pl.program_id(0)

    @pl.when(step == 0)
    def _init():
        acc_ref[...] = jnp.zeros_like(acc_ref)

    acc_ref[...] += jnp.dot(flat_ref[...], w1_ref[...],
                            preferred_element_type=jnp.float32)

    @pl.when(step == FC_STEPS - 1)
    def _finish():
        y1 = acc_ref[...] + b1_ref[...]
        y2 = jnp.dot(y1, w2_ref[...], preferred_element_type=jnp.float32) + b2_ref[...]
        y3 = jnp.dot(y2, w3_ref[...], preferred_element_type=jnp.float32) + b3_ref[...]
        out_ref[...] = y3


def _fc_head(flat, fc1_W, fc1_b, fc2_W, fc2_b, fc3_W, fc3_b):
    return pl.pallas_call(
        _fc_head_body,
        grid=(FC_STEPS,),
        in_specs=[
            pl.BlockSpec((1, FC_BK), lambda i: (0, i)),
            pl.BlockSpec((FC_BK, 256), lambda i: (i, 0)),
            pl.BlockSpec((1, 256), lambda i: (0, 0)),
            pl.BlockSpec((256, 128), lambda i: (0, 0)),
            pl.BlockSpec((1, 128), lambda i: (0, 0)),
            pl.BlockSpec((128, NC), lambda i: (0, 0)),
            pl.BlockSpec((1, NC), lambda i: (0, 0)),
        ],
        out_specs=pl.BlockSpec((1, NC), lambda i: (0, 0)),
        out_shape=jax.ShapeDtypeStruct((1, NC), jnp.float32),
        scratch_shapes=[pltpu.VMEM((1, 256), jnp.float32)],
    )(flat, fc1_W, fc1_b.reshape(1, 256), fc2_W, fc2_b.reshape(1, 128),
      fc3_W, fc3_b.reshape(1, NC))


def kernel(x, edge_index, edge_weight, conv1_W0, conv1_W1, conv1_b,
           convs_W0, convs_W1, convs_b, fc1_W, fc1_b, fc2_W, fc2_b,
           fc3_W, fc3_b):
    row, col = edge_index[0], edge_index[1]
    deg = jnp.zeros((N,), edge_weight.dtype).at[row].add(edge_weight)
    dinv = jnp.where(deg > 0.0, lax.rsqrt(deg), 0.0)
    neg_w = -(dinv[row] * edge_weight * dinv[col])

    def cheb(h, W0, W1, b):
        Tx1 = jnp.zeros_like(h).at[col].add(h[row] * neg_w[:, None])
        return h @ W0 + Tx1 @ W1 + b

    h = jax.nn.relu(cheb(x, conv1_W0, conv1_W1, conv1_b))
    for i in range(5):
        h = cheb(h, convs_W0[i], convs_W1[i], convs_b[i])
        if i < 4:
            h = jax.nn.relu(h)
    flat = h.reshape(1, -1)
    return _fc_head(flat, fc1_W, fc1_b, fc2_W, fc2_b, fc3_W, fc3_b)


# speedup vs baseline: 22.5847x; 22.5847x over previous
import functools

import jax
import jax.numpy as jnp
from jax import lax
from jax.experimental import pallas as pl
from jax.experimental.pallas import tpu as pltpu
from jax.experimental.pallas import tpu_sc as plsc

N = 10000
E = 640000
T = 50
NF = 32
NC = 2

NSC = 2
NTILES = 16
NWORK = NSC * NTILES
EPW = E // NWORK
CH = 100
NCHUNK = EPW // CH
E2D = E // CH
NP = 10240
WB_TILES = 10
WB_ROWS = N // WB_TILES

GROUPS = list(range(0, CH - 15, 16))
if GROUPS[-1] != CH - 16:
    GROUPS.append(CH - 16)

_MESH = plsc.VectorSubcoreMesh(core_axis_name="c", subcore_axis_name="s",
                               num_cores=NSC, num_subcores=NTILES)


def _splat(v16, e):
    return jnp.take_along_axis(v16, jnp.full((16,), e, jnp.int32), axis=0)


@functools.partial(
    pl.kernel,
    out_type=jax.ShapeDtypeStruct((NSC, NP), jnp.float32),
    mesh=_MESH,
    compiler_params=pltpu.CompilerParams(needs_layout_passes=False, use_tc_tiling_on_sc=False),
    scratch_types=[
        pltpu.VMEM((NCHUNK, CH), jnp.int32),
        pltpu.VMEM((NCHUNK, CH), jnp.float32),
        pltpu.MemorySpace.VMEM_SHARED((NP,), jnp.float32),
    ],
)
def _deg_kernel(row_hbm, w_hbm, zeros_hbm, out_hbm, row_v, w_v, deg_sh):
    cid = lax.axis_index("c")
    sid = lax.axis_index("s")
    wid = cid * NTILES + sid

    @pl.when(sid == 0)
    def _():
        pltpu.sync_copy(zeros_hbm, deg_sh)

    pltpu.sync_copy(row_hbm.at[pl.ds(wid * NCHUNK, NCHUNK), :], row_v)
    pltpu.sync_copy(w_hbm.at[pl.ds(wid * NCHUNK, NCHUNK), :], w_v)
    plsc.subcore_barrier()

    @pl.loop(0, NCHUNK)
    def _(j):
        pltpu.sync_copy(w_v.at[j], deg_sh.at[row_v.at[j]], add=True)

    plsc.subcore_barrier()
    pltpu.sync_copy(deg_sh.at[pl.ds(sid * 640, 640)],
                    out_hbm.at[cid, pl.ds(sid * 640, 640)])


@functools.partial(
    pl.kernel,
    out_type=jax.ShapeDtypeStruct((E2D, CH), jnp.float32),
    mesh=_MESH,
    compiler_params=pltpu.CompilerParams(needs_layout_passes=False, use_tc_tiling_on_sc=False),
    scratch_types=[
        pltpu.VMEM((NSC, NP), jnp.float32),
        pltpu.VMEM((NP,), jnp.float32),
        pltpu.VMEM((NCHUNK, CH), jnp.int32),
        pltpu.VMEM((NCHUNK, CH), jnp.int32),
        pltpu.VMEM((NCHUNK, CH), jnp.float32),
        pltpu.VMEM((NCHUNK, CH), jnp.float32),
    ],
)
def _negw_kernel(deg_hbm, row_hbm, col_hbm, w_hbm, out_hbm,
                 deg_v, dinv_v, row_v, col_v, w_v, o_v):
    cid = lax.axis_index("c")
    sid = lax.axis_index("s")
    wid = cid * NTILES + sid
    pltpu.sync_copy(deg_hbm, deg_v)
    pltpu.sync_copy(row_hbm.at[pl.ds(wid * NCHUNK, NCHUNK), :], row_v)
    pltpu.sync_copy(col_hbm.at[pl.ds(wid * NCHUNK, NCHUNK), :], col_v)
    pltpu.sync_copy(w_hbm.at[pl.ds(wid * NCHUNK, NCHUNK), :], w_v)

    @pl.loop(0, NP // 16, unroll=4)
    def _(i):
        d = deg_v[0, pl.ds(i * 16, 16)] + deg_v[1, pl.ds(i * 16, 16)]
        di = lax.bitcast_convert_type(d, jnp.int32)
        y = lax.bitcast_convert_type(
            jnp.full((16,), 0x5F3759DF, jnp.int32)
            - lax.shift_right_logical(di, 1), jnp.float32)
        xh = d * 0.5
        y = y * (1.5 - xh * y * y)
        y = y * (1.5 - xh * y * y)
        y = y * (1.5 - xh * y * y)
        dinv_v[pl.ds(i * 16, 16)] = jnp.where(d > 0.0, y, 0.0)

    @pl.loop(0, NCHUNK)
    def _(j):
        for g in GROUPS:
            r = row_v[j, pl.ds(g, 16)]
            c = col_v[j, pl.ds(g, 16)]
            w = w_v[j, pl.ds(g, 16)]
            dr = plsc.load_gather(dinv_v, [r])
            dc = plsc.load_gather(dinv_v, [c])
            o_v[j, pl.ds(g, 16)] = -(dr * w * dc)

    pltpu.sync_copy(o_v, out_hbm.at[pl.ds(wid * NCHUNK, NCHUNK), :])


@functools.partial(
    pl.kernel,
    out_type=jax.ShapeDtypeStruct((NSC, N, NF), jnp.float32),
    mesh=_MESH,
    compiler_params=pltpu.CompilerParams(needs_layout_passes=False, use_tc_tiling_on_sc=False),
    scratch_types=[
        pltpu.VMEM((NCHUNK, CH), jnp.int32),
        pltpu.VMEM((NCHUNK, CH), jnp.int32),
        pltpu.VMEM((NCHUNK, CH), jnp.float32),
        pltpu.VMEM((CH, NF), jnp.float32),
        pltpu.VMEM((CH, NF), jnp.float32),
        pltpu.MemorySpace.VMEM_SHARED((N, NF), jnp.float32),
    ],
)
def _scatter_kernel(b_hbm, row_hbm, col_hbm, w_hbm, zeros_hbm, out_hbm,
                    row_v, col_v, w_v, rows_v, msgs_v, acc_sh):
    cid = lax.axis_index("c")
    sid = lax.axis_index("s")
    wid = cid * NTILES + sid

    @pl.when(sid == 0)
    def _():
        pltpu.sync_copy(zeros_hbm, acc_sh)

    pltpu.sync_copy(row_hbm.at[pl.ds(wid * NCHUNK, NCHUNK), :], row_v)
    pltpu.sync_copy(col_hbm.at[pl.ds(wid * NCHUNK, NCHUNK), :], col_v)
    pltpu.sync_copy(w_hbm.at[pl.ds(wid * NCHUNK, NCHUNK), :], w_v)
    plsc.subcore_barrier()

    @pl.loop(0, NCHUNK)
    def _(j):
        pltpu.sync_copy(b_hbm.at[row_v.at[j]], rows_v)
        for g in GROUPS:
            w16 = w_v[j, pl.ds(g, 16)]
            for e in range(16):
                ei = g + e
                ws = _splat(w16, e)
                msgs_v[ei, pl.ds(0, 16)] = rows_v[ei, pl.ds(0, 16)] * ws
                msgs_v[ei, pl.ds(16, 16)] = rows_v[ei, pl.ds(16, 16)] * ws
        pltpu.sync_copy(msgs_v, acc_sh.at[col_v.at[j]], add=True)

    plsc.subcore_barrier()

    @pl.when(sid < WB_TILES)
    def _():
        pltpu.sync_copy(acc_sh.at[pl.ds(sid * WB_ROWS, WB_ROWS), :],
                        out_hbm.at[cid, pl.ds(sid * WB_ROWS, WB_ROWS), :])


def _mm2_body(h_ref, w0_ref, w1_ref, b_ref, a_ref, bm_ref):
    h = h_ref[...]
    a_ref[...] = jnp.dot(h, w0_ref[...], preferred_element_type=jnp.float32) + b_ref[...]
    bm_ref[...] = jnp.dot(h, w1_ref[...], preferred_element_type=jnp.float32)


def _mm2(h, W0, W1, b):
    return pl.pallas_call(
        _mm2_body,
        out_shape=(jax.ShapeDtypeStruct((N, NF), jnp.float32),
                   jax.ShapeDtypeStruct((N, NF), jnp.float32)),
    )(h, W0, W1, b.reshape(1, NF))


def _comb_body(a_ref, p_ref, w0_ref, w1_ref, b_ref, a2_ref, bm_ref):
    h = jnp.maximum(a_ref[...] + p_ref[0] + p_ref[1], 0.0)
    a2_ref[...] = jnp.dot(h, w0_ref[...], preferred_element_type=jnp.float32) + b_ref[...]
    bm_ref[...] = jnp.dot(h, w1_ref[...], preferred_element_type=jnp.float32)


def _comb(a, p, W0, W1, b):
    return pl.pallas_call(
        _comb_body,
        out_shape=(jax.ShapeDtypeStruct((N, NF), jnp.float32),
                   jax.ShapeDtypeStruct((N, NF), jnp.float32)),
    )(a, p, W0, W1, b.reshape(1, NF))


def _final_body(a_ref, p_ref, o_ref):
    o_ref[...] = a_ref[...] + p_ref[0] + p_ref[1]


def _final(a, p):
    return pl.pallas_call(
        _final_body,
        out_shape=jax.ShapeDtypeStruct((N, NF), jnp.float32),
    )(a, p)


FC_BK = 6400
FC_STEPS = (NF * N) // FC_BK


def _fc_head_body(flat_ref, w1_ref, b1_ref, w2_ref, b2_ref, w3_ref, b3_ref,
                  out_ref, acc_ref):
    step = pl.program_id(0)

    @pl.when(step == 0)
    def _init():
        acc_ref[...] = jnp.zeros_like(acc_ref)

    acc_ref[...] += jnp.dot(flat_ref[...], w1_ref[...],
                            preferred_element_type=jnp.float32)

    @pl.when(step == FC_STEPS - 1)
    def _finish():
        y1 = acc_ref[...] + b1_ref[...]
        y2 = jnp.dot(y1, w2_ref[...], preferred_element_type=jnp.float32) + b2_ref[...]
        y3 = jnp.dot(y2, w3_ref[...], preferred_element_type=jnp.float32) + b3_ref[...]
        out_ref[...] = y3


def _fc_head(flat, fc1_W, fc1_b, fc2_W, fc2_b, fc3_W, fc3_b):
    return pl.pallas_call(
        _fc_head_body,
        grid=(FC_STEPS,),
        in_specs=[
            pl.BlockSpec((1, FC_BK), lambda i: (0, i)),
            pl.BlockSpec((FC_BK, 256), lambda i: (i, 0)),
            pl.BlockSpec((1, 256), lambda i: (0, 0)),
            pl.BlockSpec((256, 128), lambda i: (0, 0)),
            pl.BlockSpec((1, 128), lambda i: (0, 0)),
            pl.BlockSpec((128, NC), lambda i: (0, 0)),
            pl.BlockSpec((1, NC), lambda i: (0, 0)),
        ],
        out_specs=pl.BlockSpec((1, NC), lambda i: (0, 0)),
        out_shape=jax.ShapeDtypeStruct((1, NC), jnp.float32),
        scratch_shapes=[pltpu.VMEM((1, 256), jnp.float32)],
    )(flat, fc1_W, fc1_b.reshape(1, 256), fc2_W, fc2_b.reshape(1, 128),
      fc3_W, fc3_b.reshape(1, NC))


def kernel(x, edge_index, edge_weight, conv1_W0, conv1_W1, conv1_b,
           convs_W0, convs_W1, convs_b, fc1_W, fc1_b, fc2_W, fc2_b,
           fc3_W, fc3_b):
    row_r = edge_index[0].reshape(E2D, CH)
    col_r = edge_index[1].reshape(E2D, CH)
    w_r = edge_weight.reshape(E2D, CH)
    zeros_np = jnp.zeros((NP,), jnp.float32)
    zeros_nf = jnp.zeros((N, NF), jnp.float32)

    deg = _deg_kernel(row_r, w_r, zeros_np)
    negw = _negw_kernel(deg, row_r, col_r, w_r)

    a, b = _mm2(x, conv1_W0, conv1_W1, conv1_b)
    for i in range(6):
        p = _scatter_kernel(b, row_r, col_r, negw, zeros_nf)
        if i < 5:
            a, b = _comb(a, p, convs_W0[i], convs_W1[i], convs_b[i])
        else:
            h6 = _final(a, p)

    flat = h6.reshape(1, NF * N)
    return _fc_head(flat, fc1_W, fc1_b, fc2_W, fc2_b, fc3_W, fc3_b)


# speedup vs baseline: 45.8017x; 2.0280x over previous
import functools

import jax
import jax.numpy as jnp
from jax import lax
from jax.experimental import pallas as pl
from jax.experimental.pallas import tpu as pltpu
from jax.experimental.pallas import tpu_sc as plsc

N = 10000
E = 640000
T = 50
NF = 32
NC = 2
TP = 64

NSC = 2
NTILES = 16
NWORK = NSC * NTILES
EPW = E // NWORK
CH = 100
NCHUNK = EPW // CH
E2D = E // CH
NP = 10240
WB_TILES = 10
WB_ROWS = N // WB_TILES
NB = 4

GROUPS = list(range(0, CH - 15, 16))
if GROUPS[-1] != CH - 16:
    GROUPS.append(CH - 16)

_MESH = plsc.VectorSubcoreMesh(core_axis_name="c", subcore_axis_name="s",
                               num_cores=NSC, num_subcores=NTILES)
_SC_PARAMS = pltpu.CompilerParams(needs_layout_passes=False,
                                  use_tc_tiling_on_sc=False)


def _splat(v16, e):
    return jnp.take_along_axis(v16, jnp.full((16,), e, jnp.int32), axis=0)


@functools.partial(
    pl.kernel,
    out_type=jax.ShapeDtypeStruct((NSC, NP), jnp.float32),
    mesh=_MESH,
    compiler_params=_SC_PARAMS,
    scratch_types=[
        pltpu.VMEM((NCHUNK, CH), jnp.int32),
        pltpu.VMEM((NCHUNK, CH), jnp.float32),
        pltpu.MemorySpace.VMEM_SHARED((NP,), jnp.float32),
    ],
)
def _deg_kernel(row_hbm, w_hbm, zeros_hbm, out_hbm, row_v, w_v, deg_sh):
    cid = lax.axis_index("c")
    sid = lax.axis_index("s")
    wid = cid * NTILES + sid

    @pl.when(sid == 0)
    def _():
        pltpu.sync_copy(zeros_hbm, deg_sh)

    pltpu.sync_copy(row_hbm.at[pl.ds(wid * NCHUNK, NCHUNK), :], row_v)
    pltpu.sync_copy(w_hbm.at[pl.ds(wid * NCHUNK, NCHUNK), :], w_v)
    plsc.subcore_barrier()

    @pl.loop(0, NCHUNK)
    def _(j):
        pltpu.sync_copy(w_v.at[j], deg_sh.at[row_v.at[j]], add=True)

    plsc.subcore_barrier()
    pltpu.sync_copy(deg_sh.at[pl.ds(sid * 640, 640)],
                    out_hbm.at[cid, pl.ds(sid * 640, 640)])


@functools.partial(
    pl.kernel,
    out_type=jax.ShapeDtypeStruct((E2D, CH), jnp.float32),
    mesh=_MESH,
    compiler_params=_SC_PARAMS,
    scratch_types=[
        pltpu.VMEM((NSC, NP), jnp.float32),
        pltpu.VMEM((NP,), jnp.float32),
        pltpu.VMEM((NCHUNK, CH), jnp.int32),
        pltpu.VMEM((NCHUNK, CH), jnp.int32),
        pltpu.VMEM((NCHUNK, CH), jnp.float32),
        pltpu.VMEM((NCHUNK, CH), jnp.float32),
    ],
)
def _negw_kernel(deg_hbm, row_hbm, col_hbm, w_hbm, out_hbm,
                 deg_v, dinv_v, row_v, col_v, w_v, o_v):
    cid = lax.axis_index("c")
    sid = lax.axis_index("s")
    wid = cid * NTILES + sid
    pltpu.sync_copy(deg_hbm, deg_v)
    pltpu.sync_copy(row_hbm.at[pl.ds(wid * NCHUNK, NCHUNK), :], row_v)
    pltpu.sync_copy(col_hbm.at[pl.ds(wid * NCHUNK, NCHUNK), :], col_v)
    pltpu.sync_copy(w_hbm.at[pl.ds(wid * NCHUNK, NCHUNK), :], w_v)

    @pl.loop(0, NP // 16, unroll=4)
    def _(i):
        d = deg_v[0, pl.ds(i * 16, 16)] + deg_v[1, pl.ds(i * 16, 16)]
        di = lax.bitcast_convert_type(d, jnp.int32)
        y = lax.bitcast_convert_type(
            jnp.full((16,), 0x5F3759DF, jnp.int32)
            - lax.shift_right_logical(di, 1), jnp.float32)
        xh = d * 0.5
        y = y * (1.5 - xh * y * y)
        y = y * (1.5 - xh * y * y)
        y = y * (1.5 - xh * y * y)
        dinv_v[pl.ds(i * 16, 16)] = jnp.where(d > 0.0, y, 0.0)

    @pl.loop(0, NCHUNK)
    def _(j):
        for g in GROUPS:
            r = row_v[j, pl.ds(g, 16)]
            c = col_v[j, pl.ds(g, 16)]
            w = w_v[j, pl.ds(g, 16)]
            dr = plsc.load_gather(dinv_v, [r])
            dc = plsc.load_gather(dinv_v, [c])
            o_v[j, pl.ds(g, 16)] = -(dr * w * dc)

    pltpu.sync_copy(o_v, out_hbm.at[pl.ds(wid * NCHUNK, NCHUNK), :])


def _make_scatter(width):
    @functools.partial(
        pl.kernel,
        out_type=jax.ShapeDtypeStruct((NSC, N, width), jnp.float32),
        mesh=_MESH,
        compiler_params=_SC_PARAMS,
        scratch_types=[
            pltpu.VMEM((NCHUNK, CH), jnp.int32),
            pltpu.VMEM((NCHUNK, CH), jnp.int32),
            pltpu.VMEM((NCHUNK, CH), jnp.float32),
            pltpu.VMEM((NB, CH, width), jnp.float32),
            pltpu.VMEM((NB, CH, width), jnp.float32),
            pltpu.MemorySpace.VMEM_SHARED((N, width), jnp.float32),
        ] + [pltpu.SemaphoreType.DMA] * (2 * NB),
    )
    def _scatter_kernel(b_hbm, row_hbm, col_hbm, w_hbm, zeros_hbm, out_hbm,
                        row_v, col_v, w_v, rows_v, msgs_v, acc_sh, *sems):
        sem_g = sems[:NB]
        sem_s = sems[NB:]
        cid = lax.axis_index("c")
        sid = lax.axis_index("s")
        wid = cid * NTILES + sid

        @pl.when(sid == 0)
        def _():
            pltpu.sync_copy(zeros_hbm, acc_sh)

        pltpu.sync_copy(row_hbm.at[pl.ds(wid * NCHUNK, NCHUNK), :], row_v)
        pltpu.sync_copy(col_hbm.at[pl.ds(wid * NCHUNK, NCHUNK), :], col_v)
        pltpu.sync_copy(w_hbm.at[pl.ds(wid * NCHUNK, NCHUNK), :], w_v)
        plsc.subcore_barrier()

        for b in range(NB):
            pltpu.async_copy(b_hbm.at[row_v.at[b]], rows_v.at[b], sem_g[b])

        @pl.loop(0, NCHUNK, step=NB)
        def _(jj):
            for b in range(NB):
                j = jj + b
                pltpu.make_async_copy(b_hbm.at[row_v.at[j]], rows_v.at[b],
                                      sem_g[b]).wait()

                @pl.when(jj > 0)
                def _():
                    pltpu.make_async_copy(msgs_v.at[b],
                                          acc_sh.at[col_v.at[j - NB]],
                                          sem_s[b]).wait()

                for g in GROUPS:
                    w16 = w_v[j, pl.ds(g, 16)]
                    for e in range(16):
                        ei = g + e
                        ws = _splat(w16, e)
                        for q in range(width // 16):
                            msgs_v[b, ei, pl.ds(q * 16, 16)] = (
                                rows_v[b, ei, pl.ds(q * 16, 16)] * ws)
                pltpu.async_copy(msgs_v.at[b], acc_sh.at[col_v.at[j]],
                                 sem_s[b], add=True)

                @pl.when(j + NB < NCHUNK)
                def _():
                    pltpu.async_copy(b_hbm.at[row_v.at[j + NB]], rows_v.at[b],
                                     sem_g[b])

        for b in range(NB):
            pltpu.make_async_copy(msgs_v.at[b],
                                  acc_sh.at[col_v.at[NCHUNK - NB + b]],
                                  sem_s[b]).wait()

        plsc.subcore_barrier()

        @pl.when(sid < WB_TILES)
        def _():
            pltpu.sync_copy(acc_sh.at[pl.ds(sid * WB_ROWS, WB_ROWS), :],
                            out_hbm.at[cid, pl.ds(sid * WB_ROWS, WB_ROWS), :])

    return _scatter_kernel


_scatter32 = _make_scatter(NF)


def _make_conv(relu):
    def _body(h_ref, p_ref, w0_ref, w1_ref, b_ref, o_ref):
        h = h_ref[...].astype(jnp.bfloat16)
        tx = (p_ref[0] + p_ref[1]).astype(jnp.bfloat16)
        o = (jnp.dot(h, w0_ref[...].astype(jnp.bfloat16),
                     preferred_element_type=jnp.float32)
             + jnp.dot(tx, w1_ref[...].astype(jnp.bfloat16),
                       preferred_element_type=jnp.float32)
             + b_ref[...])
        if relu:
            o = jnp.maximum(o, 0.0)
        o_ref[...] = o

    def _call(h, p, W0, W1, b):
        return pl.pallas_call(
            _body,
            out_shape=jax.ShapeDtypeStruct((N, NF), jnp.float32),
        )(h, p, W0, W1, b.reshape(1, NF))

    return _call


_conv_mid_tc = _make_conv(True)
_conv_last_tc = _make_conv(False)


def _conv1_body(x_ref, pa_ref, pb_ref, w0_ref, w1a_ref, w1b_ref, b_ref, o_ref):
    txa = (pa_ref[0] + pa_ref[1]).astype(jnp.bfloat16)
    txb = (pb_ref[0] + pb_ref[1]).astype(jnp.bfloat16)
    o = (jnp.dot(x_ref[...].astype(jnp.bfloat16),
                 w0_ref[...].astype(jnp.bfloat16),
                 preferred_element_type=jnp.float32)
         + jnp.dot(txa, w1a_ref[...].astype(jnp.bfloat16),
                   preferred_element_type=jnp.float32)
         + jnp.dot(txb, w1b_ref[...].astype(jnp.bfloat16),
                   preferred_element_type=jnp.float32)
         + b_ref[...])
    o_ref[...] = jnp.maximum(o, 0.0)


def _conv1_tc(x, pa, pb, W0, W1a, W1b, b):
    return pl.pallas_call(
        _conv1_body,
        out_shape=jax.ShapeDtypeStruct((N, NF), jnp.float32),
    )(x, pa, pb, W0, W1a, W1b, b.reshape(1, NF))


FC_BK = 6400
FC_STEPS = (NF * N) // FC_BK


def _fc_head_body(flat_ref, w1_ref, b1_ref, w2_ref, b2_ref, w3_ref, b3_ref,
                  out_ref, acc_ref):
    step = pl.program_id(0)

    @pl.when(step == 0)
    def _init():
        acc_ref[...] = jnp.zeros_like(acc_ref)

    acc_ref[...] += jnp.dot(flat_ref[...].astype(jnp.bfloat16),
                            w1_ref[...].astype(jnp.bfloat16),
                            preferred_element_type=jnp.float32)

    @pl.when(step == FC_STEPS - 1)
    def _finish():
        y1 = acc_ref[...] + b1_ref[...]
        y2 = jnp.dot(y1.astype(jnp.bfloat16),
                     w2_ref[...].astype(jnp.bfloat16),
                     preferred_element_type=jnp.float32) + b2_ref[...]
        y3 = jnp.dot(y2.astype(jnp.bfloat16),
                     w3_ref[...].astype(jnp.bfloat16),
                     preferred_element_type=jnp.float32) + b3_ref[...]
        out_ref[...] = y3


def _fc_head(flat, fc1_W, fc1_b, fc2_W, fc2_b, fc3_W, fc3_b):
    return pl.pallas_call(
        _fc_head_body,
        grid=(FC_STEPS,),
        in_specs=[
            pl.BlockSpec((1, FC_BK), lambda i: (0, i)),
            pl.BlockSpec((FC_BK, 256), lambda i: (i, 0)),
            pl.BlockSpec((1, 256), lambda i: (0, 0)),
            pl.BlockSpec((256, 128), lambda i: (0, 0)),
            pl.BlockSpec((1, 128), lambda i: (0, 0)),
            pl.BlockSpec((128, NC), lambda i: (0, 0)),
            pl.BlockSpec((1, NC), lambda i: (0, 0)),
        ],
        out_specs=pl.BlockSpec((1, NC), lambda i: (0, 0)),
        out_shape=jax.ShapeDtypeStruct((1, NC), jnp.float32),
        scratch_shapes=[pltpu.VMEM((1, 256), jnp.float32)],
    )(flat, fc1_W, fc1_b.reshape(1, 256), fc2_W, fc2_b.reshape(1, 128),
      fc3_W, fc3_b.reshape(1, NC))


def kernel(x, edge_index, edge_weight, conv1_W0, conv1_W1, conv1_b,
           convs_W0, convs_W1, convs_b, fc1_W, fc1_b, fc2_W, fc2_b,
           fc3_W, fc3_b):
    row_r = edge_index[0].reshape(E2D, CH)
    col_r = edge_index[1].reshape(E2D, CH)
    w_r = edge_weight.reshape(E2D, CH)
    zeros_np = jnp.zeros((NP,), jnp.float32)
    zeros_32 = jnp.zeros((N, NF), jnp.float32)

    deg = _deg_kernel(row_r, w_r, zeros_np)
    negw = _negw_kernel(deg, row_r, col_r, w_r)

    xa = x[:, :NF]
    xb = jnp.pad(x[:, NF:], ((0, 0), (0, 2 * NF - T)))
    W1a = conv1_W1[:NF]
    W1b = jnp.pad(conv1_W1[NF:], ((0, 2 * NF - T), (0, 0)))
    pa = _scatter32(xa, row_r, col_r, negw, zeros_32)
    pb = _scatter32(xb, row_r, col_r, negw, zeros_32)
    h = _conv1_tc(x, pa, pb, conv1_W0, W1a, W1b, conv1_b)
    for i in range(5):
        p = _scatter32(h, row_r, col_r, negw, zeros_32)
        conv = _conv_mid_tc if i < 4 else _conv_last_tc
        h = conv(h, p, convs_W0[i], convs_W1[i], convs_b[i])

    flat = h.reshape(1, NF * N)
    return _fc_head(flat, fc1_W, fc1_b, fc2_W, fc2_b, fc3_W, fc3_b)
